# baseline (speedup 1.0000x reference)
import functools

import jax
import jax.numpy as jnp
from jax import lax
from jax.experimental import pallas as pl
from jax.experimental.pallas import tpu as pltpu
from jax.experimental.pallas import tpu_sc as plsc

_NUM_CLASSES = 100000
_NF = 26
_D = 16
_B = 16384
_NC = 2
_NS = 16
_NW = _NC * _NS
_SPW = _B // _NW
_CB = 64
_NCHUNK = _SPW // _CB
_RPC = _CB * _NF
_GROWS = 128
_GPC = _RPC // _GROWS
_IDXROWS_PW = _SPW * _NF // _GROWS

_mesh = plsc.VectorSubcoreMesh(core_axis_name="c", subcore_axis_name="s")


@functools.partial(
    pl.kernel,
    out_type=jax.ShapeDtypeStruct((_B,), jnp.float32),
    mesh=_mesh,
    compiler_params=pltpu.CompilerParams(
        needs_layout_passes=False, use_tc_tiling_on_sc=False),
    scratch_types=[
        pltpu.VMEM((_IDXROWS_PW, _GROWS), jnp.int32),
        pltpu.VMEM((_RPC, _D), jnp.float32),
        pltpu.VMEM((_RPC,), jnp.float32),
        pltpu.VMEM((_SPW,), jnp.float32),
        pltpu.VMEM((_D,), jnp.float32),
        pltpu.SemaphoreType.DMA,
        pltpu.SemaphoreType.DMA,
    ],
)
def _fm_sc(idx_hbm, quad_hbm, lin_hbm, bias_hbm, out_hbm,
           idx_v, rows_v, lin_v, out_v, bias_v, qsem, lsem):
    cid = lax.axis_index("c")
    sid = lax.axis_index("s")
    wid = sid * _NC + cid
    pltpu.sync_copy(idx_hbm.at[pl.ds(wid * _IDXROWS_PW, _IDXROWS_PW)], idx_v)
    pltpu.sync_copy(bias_hbm, bias_v)
    lanes = lax.iota(jnp.int32, _D)
    last_lane = lanes == _D - 1

    def chunk_body(ch, carry):
        g0 = ch * _GPC
        for j in range(_GPC):
            row = idx_v.at[g0 + j]
            pltpu.async_copy(quad_hbm.at[row],
                             rows_v.at[pl.ds(j * _GROWS, _GROWS)], qsem)
            pltpu.async_copy(lin_hbm.at[row],
                             lin_v.at[pl.ds(j * _GROWS, _GROWS)], lsem)
        pltpu.make_async_copy(quad_hbm.at[pl.ds(0, _RPC)], rows_v, qsem).wait()
        pltpu.make_async_copy(lin_hbm.at[pl.ds(0, _RPC)], lin_v, lsem).wait()

        def sample_body(i, carry2):
            base = i * _NF
            v = rows_v[base, :]
            s_acc = v
            q_acc = v * v
            for f in range(1, _NF):
                v = rows_v[base + f, :]
                s_acc = s_acc + v
                q_acc = q_acc + v * v
            a = lin_v[pl.ds(base, _D)]
            b = lin_v[pl.ds(base + _NF - _D, _D)]
            pre = (0.5 * (s_acc * s_acc - q_acc) + a
                   + jnp.where(lanes >= 2 * _D - _NF, b, 0.0))
            cum = plsc.cumsum(pre)
            tgt = jnp.broadcast_to(ch * _CB + i, (_D,))
            plsc.store_scatter(out_v, [tgt], cum, mask=last_lane)
            return carry2

        lax.fori_loop(0, _CB, sample_body, 0)
        return carry

    lax.fori_loop(0, _NCHUNK, chunk_body, 0)

    def sig_body(k, carry):
        x = out_v[pl.ds(k * _D, _D)] + bias_v[...]
        out_v[pl.ds(k * _D, _D)] = 1.0 / (1.0 + jnp.exp(-x))
        return carry

    lax.fori_loop(0, _SPW // _D, sig_body, 0)
    pltpu.sync_copy(out_v, out_hbm.at[pl.ds(wid * _SPW, _SPW)])


def kernel(input, quad_table, linear_table, global_bias):
    offsets = jnp.arange(_NF, dtype=jnp.int32) * _NUM_CLASSES
    idx = (input + offsets[None, :]).reshape(_B * _NF // _GROWS, _GROWS)
    lin_flat = linear_table.reshape(-1)
    bias16 = jnp.broadcast_to(global_bias.astype(jnp.float32), (_D,))
    return _fm_sc(idx, quad_table, lin_flat, bias16)
